# baseline (speedup 1.0000x reference)
import functools
import math

import jax
import jax.numpy as jnp
from jax import lax
from jax.experimental import pallas as pl
from jax.experimental.pallas import tpu as pltpu
from jax.experimental.pallas import tpu_sc as plsc

SIZE = 32000
PADDING_INDEX = 0
SMOOTHING = 0.1
CONFIDENCE = 1.0 - SMOOTHING
EPS = SMOOTHING / (SIZE - 2)
ROW_CONST = SMOOTHING * math.log(EPS) + CONFIDENCE * math.log(CONFIDENCE)

N_ROWS = 8192

R_TC = 4096

BR = 128
RT = R_TC // BR


def _tc_body(x_ref, t_ref, w_ref, out_ref):
    i = pl.program_id(0)

    @pl.when(i == 0)
    def _():
        out_ref[0, 0] = 0.0
        out_ref[0, 1] = 0.0
        out_ref[0, 2] = 0.0

    x = x_ref[...]
    t = t_ref[:, 0]
    w = w_ref[:, 0]
    rowsum = jnp.sum(x, axis=1)
    cols = lax.broadcasted_iota(jnp.int32, (BR, SIZE), 1)
    xt = jnp.sum(jnp.where(cols == t[:, None], x, 0.0), axis=1)
    out_ref[0, 0] += jnp.sum((rowsum - x[:, 0]) * w)
    out_ref[0, 1] += jnp.sum(xt * w)
    out_ref[0, 2] += jnp.sum(w)


def _tc_part(prediction, target2, valid2):
    return pl.pallas_call(
        _tc_body,
        grid=(RT,),
        in_specs=[
            pl.BlockSpec((BR, SIZE), lambda i: (i, 0)),
            pl.BlockSpec((BR, 1), lambda i: (i, 0)),
            pl.BlockSpec((BR, 1), lambda i: (i, 0)),
        ],
        out_specs=pl.BlockSpec(memory_space=pltpu.SMEM),
        out_shape=jax.ShapeDtypeStruct((1, 3), jnp.float32),
    )(prediction, target2, valid2)


NR_SC = N_ROWS - R_TC
NRW = NR_SC // 32
RED_UNROLL = 8


NBUF = 4


@functools.partial(
    pl.kernel,
    mesh=plsc.VectorSubcoreMesh(core_axis_name="c", subcore_axis_name="s"),
    out_type=[
        jax.ShapeDtypeStruct((32, 16), jnp.float32),
        jax.ShapeDtypeStruct((32, 16), jnp.float32),
        jax.ShapeDtypeStruct((32, 16), jnp.float32),
    ],
    scratch_types=[
        pltpu.VMEM((NBUF, SIZE), jnp.float32),
        pltpu.VMEM((NRW + 16,), jnp.float32),
        pltpu.VMEM((NRW + 16,), jnp.int32),
        pltpu.VMEM((16,), jnp.float32),
        pltpu.SemaphoreType.DMA,
        pltpu.SemaphoreType.DMA,
        pltpu.SemaphoreType.DMA,
        pltpu.SemaphoreType.DMA,
    ],
)
def _sc_part(pred_hbm, wv_hbm, tgt_hbm, a_hbm, t3_hbm, n_hbm,
             row_buf, wv_v, tgt_v, stage, sem0, sem1, sem2, sem3):
    cid = lax.axis_index("c")
    sid = lax.axis_index("s")
    wid = sid * 2 + cid
    base = pl.multiple_of(R_TC + wid * NRW, 8)

    pltpu.sync_copy(wv_hbm.at[pl.ds(base, NRW)], wv_v.at[pl.ds(0, NRW)])
    pltpu.sync_copy(tgt_hbm.at[pl.ds(base, NRW)], tgt_v.at[pl.ds(0, NRW)])

    sems = (sem0, sem1, sem2, sem3)
    for b in range(NBUF):
        pltpu.async_copy(pred_hbm.at[base + b], row_buf.at[b], sems[b])

    lanes = lax.iota(jnp.int32, 16)
    lane0 = lanes == 0

    def row_chunk(j, carry):
        total, t3v, cntv = carry
        for b in range(NBUF):
            r = j * NBUF + b
            pltpu.make_async_copy(
                pred_hbm.at[0], row_buf.at[b], sems[b]).wait()

            def red(k, accs):
                off = k * (RED_UNROLL * 16)
                return tuple(
                    accs[u] + row_buf[b, pl.ds(off + u * 16, 16)]
                    for u in range(RED_UNROLL))

            accs = lax.fori_loop(
                0, SIZE // (RED_UNROLL * 16), red,
                tuple(jnp.zeros((16,), jnp.float32)
                      for _ in range(RED_UNROLL)))
            acc = ((accs[0] + accs[1]) + (accs[2] + accs[3])) + \
                  ((accs[4] + accs[5]) + (accs[6] + accs[7]))
            acc = acc - jnp.where(lane0, row_buf[b, pl.ds(0, 16)], 0.0)
            w_vec = wv_v[pl.ds(r, 16)]
            t_vec = tgt_v[pl.ds(r, 16)]
            t0 = t_vec[0]
            toff = (t0 // 16) * 16
            trel = t0 - toff
            twin = row_buf[b, pl.ds(toff, 16)]
            val = jnp.where(lanes == trel, twin, 0.0)
            total = total + acc * w_vec[0]
            t3v = t3v + val * w_vec[0]
            cntv = cntv + jnp.where(lane0, w_vec, 0.0)
            nxt = jnp.minimum(base + r + NBUF, N_ROWS - 1)
            pltpu.async_copy(pred_hbm.at[nxt], row_buf.at[b], sems[b])
        return (total, t3v, cntv)

    z = jnp.zeros((16,), jnp.float32)
    total, t3v, cntv = lax.fori_loop(0, NRW // NBUF, row_chunk, (z, z, z))
    for b in range(NBUF):
        pltpu.make_async_copy(pred_hbm.at[0], row_buf.at[b], sems[b]).wait()

    stage[...] = total
    pltpu.sync_copy(stage, a_hbm.at[wid])
    stage[...] = t3v
    pltpu.sync_copy(stage, t3_hbm.at[wid])
    stage[...] = cntv
    pltpu.sync_copy(stage, n_hbm.at[wid])


def kernel(prediction, target):
    target = target.astype(jnp.int32)
    valid = (target != PADDING_INDEX).astype(jnp.float32)

    tc_out = _tc_part(prediction, target[:, None], valid[:, None])
    a_sc, t3_sc, n_sc = _sc_part(prediction, valid, target)

    a = tc_out[0, 0] + jnp.sum(a_sc)
    t3 = tc_out[0, 1] + jnp.sum(t3_sc)
    n = tc_out[0, 2] + jnp.sum(n_sc)

    return n * ROW_CONST - EPS * a - (CONFIDENCE - EPS) * t3


# speedup vs baseline: 1.0419x; 1.0419x over previous
import functools
import math

import jax
import jax.numpy as jnp
from jax import lax
from jax.experimental import pallas as pl
from jax.experimental.pallas import tpu as pltpu
from jax.experimental.pallas import tpu_sc as plsc

SIZE = 32000
PADDING_INDEX = 0
SMOOTHING = 0.1
CONFIDENCE = 1.0 - SMOOTHING
EPS = SMOOTHING / (SIZE - 2)
ROW_CONST = SMOOTHING * math.log(EPS) + CONFIDENCE * math.log(CONFIDENCE)

N_ROWS = 8192

R_TC = 4096

BR = 128
RT = R_TC // BR


def _tc_body(x_ref, t_ref, out_ref):
    i = pl.program_id(0)

    @pl.when(i == 0)
    def _():
        out_ref[0, 0] = 0.0
        out_ref[0, 1] = 0.0
        out_ref[0, 2] = 0.0

    x = x_ref[...]
    t = t_ref[:, 0]
    w = (t != PADDING_INDEX).astype(jnp.float32)
    rowsum = jnp.sum(x, axis=1)
    cols = lax.broadcasted_iota(jnp.int32, (BR, SIZE), 1)
    xt = jnp.sum(jnp.where(cols == t[:, None], x, 0.0), axis=1)
    out_ref[0, 0] += jnp.sum((rowsum - x[:, 0]) * w)
    out_ref[0, 1] += jnp.sum(xt * w)
    out_ref[0, 2] += jnp.sum(w)


def _tc_part(prediction, target2):
    return pl.pallas_call(
        _tc_body,
        grid=(RT,),
        in_specs=[
            pl.BlockSpec((BR, SIZE), lambda i: (i, 0)),
            pl.BlockSpec((BR, 1), lambda i: (i, 0)),
        ],
        out_specs=pl.BlockSpec(memory_space=pltpu.SMEM),
        out_shape=jax.ShapeDtypeStruct((1, 3), jnp.float32),
    )(prediction, target2)


NR_SC = N_ROWS - R_TC
NRW = NR_SC // 32
RED_UNROLL = 8


NBUF = 2


@functools.partial(
    pl.kernel,
    mesh=plsc.VectorSubcoreMesh(core_axis_name="c", subcore_axis_name="s"),
    out_type=[
        jax.ShapeDtypeStruct((32, 16), jnp.float32),
        jax.ShapeDtypeStruct((32, 16), jnp.float32),
        jax.ShapeDtypeStruct((32, 16), jnp.float32),
    ],
    scratch_types=[
        pltpu.VMEM((NBUF, SIZE), jnp.float32),
        pltpu.VMEM((NRW + 16,), jnp.int32),
        pltpu.VMEM((16,), jnp.float32),
        pltpu.SemaphoreType.DMA,
        pltpu.SemaphoreType.DMA,
    ],
)
def _sc_part(pred_hbm, tgt_hbm, a_hbm, t3_hbm, n_hbm,
             row_buf, tgt_v, stage, sem0, sem1):
    cid = lax.axis_index("c")
    sid = lax.axis_index("s")
    wid = sid * 2 + cid
    base = pl.multiple_of(R_TC + wid * NRW, 8)

    pltpu.sync_copy(tgt_hbm.at[pl.ds(base, NRW)], tgt_v.at[pl.ds(0, NRW)])

    sems = (sem0, sem1)
    for b in range(NBUF):
        pltpu.async_copy(pred_hbm.at[base + b], row_buf.at[b], sems[b])

    lanes = lax.iota(jnp.int32, 16)
    lane0 = lanes == 0

    def row_chunk(j, carry):
        total, t3v, cntv = carry
        for b in range(NBUF):
            r = j * NBUF + b
            pltpu.make_async_copy(
                pred_hbm.at[0], row_buf.at[b], sems[b]).wait()

            def red(k, accs):
                off = k * (RED_UNROLL * 16)
                return tuple(
                    accs[u] + row_buf[b, pl.ds(off + u * 16, 16)]
                    for u in range(RED_UNROLL))

            accs = lax.fori_loop(
                0, SIZE // (RED_UNROLL * 16), red,
                tuple(jnp.zeros((16,), jnp.float32)
                      for _ in range(RED_UNROLL)))
            acc = ((accs[0] + accs[1]) + (accs[2] + accs[3])) + \
                  ((accs[4] + accs[5]) + (accs[6] + accs[7]))
            acc = acc - jnp.where(lane0, row_buf[b, pl.ds(0, 16)], 0.0)
            t_vec = tgt_v[pl.ds(r, 16)]
            w_vec = jnp.where(t_vec != PADDING_INDEX, 1.0, 0.0)
            t0 = t_vec[0]
            toff = (t0 // 16) * 16
            trel = t0 - toff
            twin = row_buf[b, pl.ds(toff, 16)]
            val = jnp.where(lanes == trel, twin, 0.0)
            total = total + acc * w_vec[0]
            t3v = t3v + val * w_vec[0]
            cntv = cntv + jnp.where(lane0, w_vec, 0.0)
            nxt = jnp.minimum(base + r + NBUF, N_ROWS - 1)
            pltpu.async_copy(pred_hbm.at[nxt], row_buf.at[b], sems[b])
        return (total, t3v, cntv)

    z = jnp.zeros((16,), jnp.float32)
    total, t3v, cntv = lax.fori_loop(0, NRW // NBUF, row_chunk, (z, z, z))
    for b in range(NBUF):
        pltpu.make_async_copy(pred_hbm.at[0], row_buf.at[b], sems[b]).wait()

    stage[...] = total
    pltpu.sync_copy(stage, a_hbm.at[wid])
    stage[...] = t3v
    pltpu.sync_copy(stage, t3_hbm.at[wid])
    stage[...] = cntv
    pltpu.sync_copy(stage, n_hbm.at[wid])


def kernel(prediction, target):
    target = target.astype(jnp.int32)

    tc_out = _tc_part(prediction, target[:, None])
    a_sc, t3_sc, n_sc = _sc_part(prediction, target)

    a = tc_out[0, 0] + jnp.sum(a_sc)
    t3 = tc_out[0, 1] + jnp.sum(t3_sc)
    n = tc_out[0, 2] + jnp.sum(n_sc)

    return n * ROW_CONST - EPS * a - (CONFIDENCE - EPS) * t3


# speedup vs baseline: 1.0593x; 1.0166x over previous
import functools
import math

import jax
import jax.numpy as jnp
from jax import lax
from jax.experimental import pallas as pl
from jax.experimental.pallas import tpu as pltpu
from jax.experimental.pallas import tpu_sc as plsc

SIZE = 32000
PADDING_INDEX = 0
SMOOTHING = 0.1
CONFIDENCE = 1.0 - SMOOTHING
EPS = SMOOTHING / (SIZE - 2)
ROW_CONST = SMOOTHING * math.log(EPS) + CONFIDENCE * math.log(CONFIDENCE)

N_ROWS = 8192

R_TC = 4096

BR = 128
RT = R_TC // BR


def _tc_body(x_ref, t_ref, out_ref):
    i = pl.program_id(0)

    @pl.when(i == 0)
    def _():
        out_ref[0, 0] = 0.0

    x = x_ref[...]
    t = t_ref[:, 0]
    w = (t != PADDING_INDEX).astype(jnp.float32)
    rowsum = jnp.sum(x, axis=1)
    cols = lax.broadcasted_iota(jnp.int32, (BR, SIZE), 1)
    xt = jnp.sum(jnp.where(cols == t[:, None], x, 0.0), axis=1)
    out_ref[0, 0] += (ROW_CONST * jnp.sum(w)
                      - EPS * jnp.sum((rowsum - x[:, 0]) * w)
                      - (CONFIDENCE - EPS) * jnp.sum(xt * w))


def _tc_part(prediction, target2):
    return pl.pallas_call(
        _tc_body,
        grid=(RT,),
        in_specs=[
            pl.BlockSpec((BR, SIZE), lambda i: (i, 0)),
            pl.BlockSpec((BR, 1), lambda i: (i, 0)),
        ],
        out_specs=pl.BlockSpec(memory_space=pltpu.SMEM),
        out_shape=jax.ShapeDtypeStruct((1, 1), jnp.float32),
    )(prediction, target2)


NR_SC = N_ROWS - R_TC
NRW = NR_SC // 32
RED_UNROLL = 8


NBUF = 2


@functools.partial(
    pl.kernel,
    mesh=plsc.VectorSubcoreMesh(core_axis_name="c", subcore_axis_name="s"),
    out_type=[
        jax.ShapeDtypeStruct((32, 16), jnp.float32),
    ],
    scratch_types=[
        pltpu.VMEM((NBUF, SIZE), jnp.float32),
        pltpu.VMEM((NRW + 16,), jnp.int32),
        pltpu.VMEM((16,), jnp.float32),
        pltpu.SemaphoreType.DMA,
        pltpu.SemaphoreType.DMA,
    ],
)
def _sc_part(pred_hbm, tgt_hbm, loss_hbm,
             row_buf, tgt_v, stage, sem0, sem1):
    cid = lax.axis_index("c")
    sid = lax.axis_index("s")
    wid = sid * 2 + cid
    base = pl.multiple_of(R_TC + wid * NRW, 8)

    pltpu.sync_copy(tgt_hbm.at[pl.ds(base, NRW)], tgt_v.at[pl.ds(0, NRW)])

    sems = (sem0, sem1)
    for b in range(NBUF):
        pltpu.async_copy(pred_hbm.at[base + b], row_buf.at[b], sems[b])

    lanes = lax.iota(jnp.int32, 16)
    lane0 = lanes == 0

    def row_chunk(j, carry):
        total, t3v, cntv = carry
        for b in range(NBUF):
            r = j * NBUF + b
            pltpu.make_async_copy(
                pred_hbm.at[0], row_buf.at[b], sems[b]).wait()

            def red(k, accs):
                off = k * (RED_UNROLL * 16)
                return tuple(
                    accs[u] + row_buf[b, pl.ds(off + u * 16, 16)]
                    for u in range(RED_UNROLL))

            accs = lax.fori_loop(
                0, SIZE // (RED_UNROLL * 16), red,
                tuple(jnp.zeros((16,), jnp.float32)
                      for _ in range(RED_UNROLL)))
            acc = ((accs[0] + accs[1]) + (accs[2] + accs[3])) + \
                  ((accs[4] + accs[5]) + (accs[6] + accs[7]))
            acc = acc - jnp.where(lane0, row_buf[b, pl.ds(0, 16)], 0.0)
            t_vec = tgt_v[pl.ds(r, 16)]
            w_vec = jnp.where(t_vec != PADDING_INDEX, 1.0, 0.0)
            t0 = t_vec[0]
            toff = (t0 // 16) * 16
            trel = t0 - toff
            twin = row_buf[b, pl.ds(toff, 16)]
            val = jnp.where(lanes == trel, twin, 0.0)
            total = total + acc * w_vec[0]
            t3v = t3v + val * w_vec[0]
            cntv = cntv + jnp.where(lane0, w_vec, 0.0)
            nxt = jnp.minimum(base + r + NBUF, N_ROWS - 1)
            pltpu.async_copy(pred_hbm.at[nxt], row_buf.at[b], sems[b])
        return (total, t3v, cntv)

    z = jnp.zeros((16,), jnp.float32)
    total, t3v, cntv = lax.fori_loop(0, NRW // NBUF, row_chunk, (z, z, z))
    for b in range(NBUF):
        pltpu.make_async_copy(pred_hbm.at[0], row_buf.at[b], sems[b]).wait()

    stage[...] = (ROW_CONST * cntv - EPS * total
                  - (CONFIDENCE - EPS) * t3v)
    pltpu.sync_copy(stage, loss_hbm.at[wid])


def kernel(prediction, target):
    target = target.astype(jnp.int32)

    tc_out = _tc_part(prediction, target[:, None])
    (sc_out,) = _sc_part(prediction, target)

    return tc_out[0, 0] + jnp.sum(sc_out)
